# baseline (speedup 1.0000x reference)
import functools

import jax
import jax.numpy as jnp
from jax import lax
from jax.experimental import pallas as pl
from jax.experimental.pallas import tpu as pltpu
from jax.experimental.pallas import tpu_sc as plsc

N = 10000
E = 320000
D_IN = 128
H = 4
D_OUT = 64
HD = H * D_OUT
CH = 64
NCH = HD // CH
NEG_SLOPE = 0.2

B = 80
NB = E // B
NBT = NB // 16
NPAIR = NBT // 2
RB = 40
NRB = N // RB
L = 16


def _node_proj_body(x_ref, wa_ref, wd_ref, xa_ref, nh_ref):
    xb = x_ref[...]
    pa = jnp.dot(xb, wa_ref[...], preferred_element_type=jnp.float32)
    pd = jnp.dot(xb, wd_ref[...], preferred_element_type=jnp.float32)
    for c in range(NCH):
        xa_ref[c] = pa[:, c * CH:(c + 1) * CH]
        nh_ref[c] = pd[:, c * CH:(c + 1) * CH]


def _node_proj(x, wa, wd):
    rbn = 2000
    return pl.pallas_call(
        _node_proj_body,
        grid=(N // rbn,),
        in_specs=[
            pl.BlockSpec((rbn, D_IN), lambda r: (r, 0)),
            pl.BlockSpec((D_IN, HD), lambda r: (0, 0)),
            pl.BlockSpec((D_IN, HD), lambda r: (0, 0)),
        ],
        out_specs=[
            pl.BlockSpec((NCH, rbn, CH), lambda r: (0, r, 0)),
            pl.BlockSpec((NCH, rbn, CH), lambda r: (0, r, 0)),
        ],
        out_shape=[
            jax.ShapeDtypeStruct((NCH, N, CH), jnp.float32),
            jax.ShapeDtypeStruct((NCH, N, CH), jnp.float32),
        ],
    )(x, wa, wd)


def _edge_proj_body(ea_ref, w_ref, out_ref):
    out_ref[...] = jnp.dot(ea_ref[...], w_ref[...],
                           preferred_element_type=jnp.float32)


def _edge_proj(edge_attr, wb):
    eb = 2000
    return pl.pallas_call(
        _edge_proj_body,
        grid=(E // eb,),
        in_specs=[
            pl.BlockSpec((eb, D_IN), lambda e: (e, 0)),
            pl.BlockSpec((D_IN, HD), lambda e: (0, 0)),
        ],
        out_specs=pl.BlockSpec((eb, HD), lambda e: (e, 0)),
        out_shape=jax.ShapeDtypeStruct((E, HD), jnp.float32),
    )(edge_attr, wb)


_mesh = plsc.VectorSubcoreMesh(core_axis_name="c", subcore_axis_name="s")


@functools.partial(
    pl.kernel,
    out_type=jax.ShapeDtypeStruct((NCH * N, CH), jnp.float32),
    mesh=_mesh,
    compiler_params=pltpu.CompilerParams(use_tc_tiling_on_sc=False),
    scratch_types=[
        pltpu.VMEM((2, 2 * B), jnp.int32),
        pltpu.VMEM((2, B), jnp.int32),
        pltpu.VMEM((2, B), jnp.int32),
        pltpu.VMEM((2, B), jnp.int32),
        pltpu.VMEM((2, B, CH), jnp.float32),
        pltpu.VMEM((2, B, CH), jnp.float32),
        pltpu.VMEM((2, B, CH), jnp.float32),
        pltpu.VMEM((RB, CH), jnp.float32),
        pltpu.VMEM((RB, CH), jnp.float32),
        pltpu.VMEM((RB, CH), jnp.float32),
        pltpu.VMEM((CH,), jnp.float32),
        pltpu.VMEM_SHARED((N, CH), jnp.float32),
        pltpu.VMEM_SHARED((N, CH), jnp.float32),
        pltpu.SemaphoreType.DMA,
        pltpu.SemaphoreType.DMA,
        pltpu.SemaphoreType.DMA,
        pltpu.SemaphoreType.DMA,
        pltpu.SemaphoreType.DMA,
        pltpu.SemaphoreType.DMA,
    ],
)
def _sc_attention(xa_hbm, nh_hbm, ea_hbm, eidx_hbm, b_hbm, out_hbm,
                  idxb, gs, gd, didx, xab, nhb, eab,
                  dbe, dbw, ob, bb, esum_sh, wsum_sh,
                  semi0, semi1, semg0, semg1, sc0, sc1):
    cid = lax.axis_index("c")
    sid = lax.axis_index("s")
    semi = (semi0, semi1)
    semg = (semg0, semg1)
    sc = (sc0, sc1)

    def start_idx(slot, t):
        bidx = sid + t * 16
        pltpu.async_copy(eidx_hbm.at[pl.ds(bidx * 2 * B, 2 * B)],
                         idxb.at[slot], semi[slot])

    def wait_idx(slot):
        pltpu.make_async_copy(eidx_hbm.at[pl.ds(0, 2 * B)],
                              idxb.at[slot], semi[slot]).wait()

    def adjust(slot, coff):
        for g in range(B // L):
            sl = pl.ds(g * L, L)
            s16 = idxb[slot, pl.ds(g * L, L)]
            d16 = idxb[slot, pl.ds(B + g * L, L)]
            gs[slot, sl] = s16 + coff
            gd[slot, sl] = d16 + coff
            didx[slot, sl] = d16

    def start_gathers(slot, t, c):
        bidx = sid + t * 16
        e0 = bidx * B
        pltpu.async_copy(xa_hbm.at[gs.at[slot]], xab.at[slot], semg[slot])
        pltpu.async_copy(nh_hbm.at[gd.at[slot]], nhb.at[slot], semg[slot])
        pltpu.async_copy(ea_hbm.at[pl.ds(e0, B), pl.ds(c * CH, CH)],
                         eab.at[slot], semg[slot])

    def wait_gathers(slot):
        pltpu.make_async_copy(xa_hbm.at[pl.ds(0, B)], xab.at[slot],
                              semg[slot]).wait()
        pltpu.make_async_copy(xa_hbm.at[pl.ds(0, B)], nhb.at[slot],
                              semg[slot]).wait()
        pltpu.make_async_copy(xa_hbm.at[pl.ds(0, B)], eab.at[slot],
                              semg[slot]).wait()

    def compute(slot):
        @plsc.parallel_loop(0, B, step=1, unroll=8)
        def _crow(r):
            for f in range(CH // L):
                sl = pl.ds(f * L, L)
                m = xab[slot, r, sl] + eab[slot, r, sl]
                t = m + nhb[slot, r, sl]
                lr = jnp.maximum(t, t * NEG_SLOPE)
                p = jnp.exp(lr)
                nhb[slot, r, sl] = p
                eab[slot, r, sl] = m * p

    def start_scatter(slot):
        pltpu.async_copy(nhb.at[slot], esum_sh.at[didx.at[slot]],
                         sc[slot], add=True)
        pltpu.async_copy(eab.at[slot], wsum_sh.at[didx.at[slot]],
                         sc[slot], add=True)

    def wait_scatter(slot):
        pltpu.make_async_copy(xa_hbm.at[pl.ds(0, B)], nhb.at[slot],
                              sc[slot]).wait()
        pltpu.make_async_copy(xa_hbm.at[pl.ds(0, B)], eab.at[slot],
                              sc[slot]).wait()

    for k in range(2):
        c = cid * 2 + k
        coff = c * N

        plsc.subcore_barrier()

        def _zrow(r, carry):
            for f in range(CH // L):
                ob[r, pl.ds(f * L, L)] = jnp.zeros((L,), jnp.float32)
            return carry
        lax.fori_loop(0, RB, _zrow, 0)

        def _zero_blk(j, carry):
            blk = sid + j * 16

            @pl.when(blk < NRB)
            def _():
                r0 = blk * RB
                pltpu.sync_copy(ob, esum_sh.at[pl.ds(r0, RB)])
                pltpu.sync_copy(ob, wsum_sh.at[pl.ds(r0, RB)])
            return carry
        lax.fori_loop(0, (NRB + 15) // 16, _zero_blk, 0)

        pltpu.sync_copy(b_hbm.at[pl.ds(c * CH, CH)], bb)
        plsc.subcore_barrier()

        start_idx(0, 0)
        start_idx(1, 1)
        wait_idx(0)
        adjust(0, coff)
        start_gathers(0, 0, c)

        def _pair(j, carry):
            for b in (0, 1):
                t = 2 * j + b
                s = 1 - b

                @pl.when(t <= NBT - 2)
                def _():
                    wait_idx(s)

                    @pl.when(t >= 1)
                    def _():
                        wait_scatter(s)
                    adjust(s, coff)
                    start_gathers(s, t + 1, c)

                    @pl.when(t <= NBT - 3)
                    def _():
                        start_idx(b, t + 2)
                wait_gathers(b)
                compute(b)
                start_scatter(b)
            return carry
        lax.fori_loop(0, NPAIR, _pair, 0)
        wait_scatter(0)
        wait_scatter(1)

        plsc.subcore_barrier()

        def _flush_blk(j, carry):
            blk = sid + j * 16

            @pl.when(blk < NRB)
            def _():
                r0 = blk * RB
                pltpu.sync_copy(esum_sh.at[pl.ds(r0, RB)], dbe)
                pltpu.sync_copy(wsum_sh.at[pl.ds(r0, RB)], dbw)

                @plsc.parallel_loop(0, RB, step=1, unroll=4)
                def _drow(r):
                    for f in range(CH // L):
                        sl = pl.ds(f * L, L)
                        es = dbe[r, sl]
                        ws = dbw[r, sl]
                        val = jnp.where(es > 0.0, ws / es, 0.0) + bb[sl]
                        ob[r, sl] = val
                pltpu.sync_copy(ob, out_hbm.at[pl.ds(coff + r0, RB)])
            return carry
        lax.fori_loop(0, (NRB + 15) // 16, _flush_blk, 0)


def kernel(x, edge_index, edge_attr, W_src, W_dst, b):
    wa = W_src[:D_IN]
    wb = W_src[D_IN:]
    xa_f, nh_f = _node_proj(x, wa, W_dst)
    ea_p = _edge_proj(edge_attr, wb)
    eidx = edge_index.reshape(2, NB, B).transpose(1, 0, 2).reshape(-1)
    out_k = _sc_attention(
        xa_f.reshape(NCH * N, CH),
        nh_f.reshape(NCH * N, CH),
        ea_p,
        eidx,
        b,
    )
    return out_k.reshape(NCH, N, D_OUT).transpose(1, 0, 2)


# speedup vs baseline: 1.0431x; 1.0431x over previous
import functools

import jax
import jax.numpy as jnp
from jax import lax
from jax.experimental import pallas as pl
from jax.experimental.pallas import tpu as pltpu
from jax.experimental.pallas import tpu_sc as plsc

N = 10000
E = 320000
D_IN = 128
H = 4
D_OUT = 64
HD = H * D_OUT
CH = 64
NCH = HD // CH
NEG_SLOPE = 0.2

B = 80
NB = E // B
NBT = NB // 16
NPAIR = NBT // 2
RB = 40
NRB = N // RB
L = 16


def _node_proj_body(x_ref, wa_ref, wd_ref, xa_ref, nh_ref):
    xb = x_ref[...]
    pa = jnp.dot(xb, wa_ref[...], preferred_element_type=jnp.float32)
    pd = jnp.dot(xb, wd_ref[...], preferred_element_type=jnp.float32)
    for c in range(NCH):
        xa_ref[c] = pa[:, c * CH:(c + 1) * CH]
        nh_ref[c] = pd[:, c * CH:(c + 1) * CH]


def _node_proj(x, wa, wd):
    rbn = 2000
    return pl.pallas_call(
        _node_proj_body,
        grid=(N // rbn,),
        in_specs=[
            pl.BlockSpec((rbn, D_IN), lambda r: (r, 0)),
            pl.BlockSpec((D_IN, HD), lambda r: (0, 0)),
            pl.BlockSpec((D_IN, HD), lambda r: (0, 0)),
        ],
        out_specs=[
            pl.BlockSpec((NCH, rbn, CH), lambda r: (0, r, 0)),
            pl.BlockSpec((NCH, rbn, CH), lambda r: (0, r, 0)),
        ],
        out_shape=[
            jax.ShapeDtypeStruct((NCH, N, CH), jnp.float32),
            jax.ShapeDtypeStruct((NCH, N, CH), jnp.float32),
        ],
    )(x, wa, wd)


def _edge_proj_body(ea_ref, w_ref, out_ref):
    out_ref[...] = jnp.dot(ea_ref[...], w_ref[...],
                           preferred_element_type=jnp.float32)


def _edge_proj(edge_attr, wb):
    eb = 2000
    return pl.pallas_call(
        _edge_proj_body,
        grid=(E // eb,),
        in_specs=[
            pl.BlockSpec((eb, D_IN), lambda e: (e, 0)),
            pl.BlockSpec((D_IN, HD), lambda e: (0, 0)),
        ],
        out_specs=pl.BlockSpec((eb, HD), lambda e: (e, 0)),
        out_shape=jax.ShapeDtypeStruct((E, HD), jnp.float32),
    )(edge_attr, wb)


_mesh = plsc.VectorSubcoreMesh(core_axis_name="c", subcore_axis_name="s")


@functools.partial(
    pl.kernel,
    out_type=jax.ShapeDtypeStruct((N, HD), jnp.float32),
    mesh=_mesh,
    compiler_params=pltpu.CompilerParams(use_tc_tiling_on_sc=False),
    scratch_types=[
        pltpu.VMEM((2, 2 * B), jnp.int32),
        pltpu.VMEM((2, B), jnp.int32),
        pltpu.VMEM((2, B), jnp.int32),
        pltpu.VMEM((2, B), jnp.int32),
        pltpu.VMEM((2, B, CH), jnp.float32),
        pltpu.VMEM((2, B, CH), jnp.float32),
        pltpu.VMEM((2, B, CH), jnp.float32),
        pltpu.VMEM((RB, CH), jnp.float32),
        pltpu.VMEM((RB, CH), jnp.float32),
        pltpu.VMEM((RB, CH), jnp.float32),
        pltpu.VMEM((CH,), jnp.float32),
        pltpu.VMEM_SHARED((N, CH), jnp.float32),
        pltpu.VMEM_SHARED((N, CH), jnp.float32),
        pltpu.SemaphoreType.DMA,
        pltpu.SemaphoreType.DMA,
        pltpu.SemaphoreType.DMA,
        pltpu.SemaphoreType.DMA,
        pltpu.SemaphoreType.DMA,
        pltpu.SemaphoreType.DMA,
    ],
)
def _sc_attention(xa_hbm, nh_hbm, ea_hbm, eidx_hbm, b_hbm, out_hbm,
                  idxb, gs, gd, didx, xab, nhb, eab,
                  dbe, dbw, ob, bb, esum_sh, wsum_sh,
                  semi0, semi1, semg0, semg1, sc0, sc1):
    cid = lax.axis_index("c")
    sid = lax.axis_index("s")
    semi = (semi0, semi1)
    semg = (semg0, semg1)
    sc = (sc0, sc1)

    def start_idx(slot, t):
        bidx = sid + t * 16
        pltpu.async_copy(eidx_hbm.at[pl.ds(bidx * 2 * B, 2 * B)],
                         idxb.at[slot], semi[slot])

    def wait_idx(slot):
        pltpu.make_async_copy(eidx_hbm.at[pl.ds(0, 2 * B)],
                              idxb.at[slot], semi[slot]).wait()

    def adjust(slot, coff):
        for g in range(B // L):
            sl = pl.ds(g * L, L)
            s16 = idxb[slot, pl.ds(g * L, L)]
            d16 = idxb[slot, pl.ds(B + g * L, L)]
            gs[slot, sl] = s16 + coff
            gd[slot, sl] = d16 + coff
            didx[slot, sl] = d16

    def start_gathers(slot, t, c):
        bidx = sid + t * 16
        e0 = bidx * B
        pltpu.async_copy(xa_hbm.at[gs.at[slot]], xab.at[slot], semg[slot])
        pltpu.async_copy(nh_hbm.at[gd.at[slot]], nhb.at[slot], semg[slot])
        pltpu.async_copy(ea_hbm.at[pl.ds(e0, B), pl.ds(c * CH, CH)],
                         eab.at[slot], semg[slot])

    def wait_gathers(slot):
        pltpu.make_async_copy(xa_hbm.at[pl.ds(0, B)], xab.at[slot],
                              semg[slot]).wait()
        pltpu.make_async_copy(xa_hbm.at[pl.ds(0, B)], nhb.at[slot],
                              semg[slot]).wait()
        pltpu.make_async_copy(xa_hbm.at[pl.ds(0, B)], eab.at[slot],
                              semg[slot]).wait()

    def compute(slot):
        @plsc.parallel_loop(0, B, step=1, unroll=4)
        def _crow(r):
            for f in range(CH // L):
                sl = pl.ds(f * L, L)
                m = xab[slot, r, sl] + eab[slot, r, sl]
                t = m + nhb[slot, r, sl]
                lr = jnp.maximum(t, t * NEG_SLOPE)
                p = jnp.exp(lr)
                nhb[slot, r, sl] = p
                eab[slot, r, sl] = m * p

    def start_scatter(slot):
        pltpu.async_copy(nhb.at[slot], esum_sh.at[didx.at[slot]],
                         sc[slot], add=True)
        pltpu.async_copy(eab.at[slot], wsum_sh.at[didx.at[slot]],
                         sc[slot], add=True)

    def wait_scatter(slot):
        pltpu.make_async_copy(xa_hbm.at[pl.ds(0, B)], nhb.at[slot],
                              sc[slot]).wait()
        pltpu.make_async_copy(xa_hbm.at[pl.ds(0, B)], eab.at[slot],
                              sc[slot]).wait()

    for k in range(2):
        c = cid * 2 + k
        coff = c * N

        plsc.subcore_barrier()

        def _zrow(r, carry):
            for f in range(CH // L):
                ob[r, pl.ds(f * L, L)] = jnp.zeros((L,), jnp.float32)
            return carry
        lax.fori_loop(0, RB, _zrow, 0)

        def _zero_blk(j, carry):
            blk = sid + j * 16

            @pl.when(blk < NRB)
            def _():
                r0 = blk * RB
                pltpu.sync_copy(ob, esum_sh.at[pl.ds(r0, RB)])
                pltpu.sync_copy(ob, wsum_sh.at[pl.ds(r0, RB)])
            return carry
        lax.fori_loop(0, (NRB + 15) // 16, _zero_blk, 0)

        pltpu.sync_copy(b_hbm.at[pl.ds(c * CH, CH)], bb)
        plsc.subcore_barrier()

        start_idx(0, 0)
        start_idx(1, 1)
        wait_idx(0)
        adjust(0, coff)
        start_gathers(0, 0, c)

        def _pair(j, carry):
            for b in (0, 1):
                t = 2 * j + b
                s = 1 - b

                @pl.when(t <= NBT - 2)
                def _():
                    wait_idx(s)

                    @pl.when(t >= 1)
                    def _():
                        wait_scatter(s)
                    adjust(s, coff)
                    start_gathers(s, t + 1, c)

                    @pl.when(t <= NBT - 3)
                    def _():
                        start_idx(b, t + 2)
                wait_gathers(b)
                compute(b)
                start_scatter(b)
            return carry
        lax.fori_loop(0, NPAIR, _pair, 0)
        wait_scatter(0)
        wait_scatter(1)

        plsc.subcore_barrier()

        def _flush_blk(j, carry):
            blk = sid + j * 16

            @pl.when(blk < NRB)
            def _():
                r0 = blk * RB
                pltpu.sync_copy(esum_sh.at[pl.ds(r0, RB)], dbe)
                pltpu.sync_copy(wsum_sh.at[pl.ds(r0, RB)], dbw)

                @plsc.parallel_loop(0, RB, step=1, unroll=4)
                def _drow(r):
                    for f in range(CH // L):
                        sl = pl.ds(f * L, L)
                        es = dbe[r, sl]
                        ws = dbw[r, sl]
                        val = jnp.where(es > 0.0, ws / es, 0.0) + bb[sl]
                        ob[r, sl] = val
                pltpu.sync_copy(
                    ob, out_hbm.at[pl.ds(r0, RB), pl.ds(c * CH, CH)])
            return carry
        lax.fori_loop(0, (NRB + 15) // 16, _flush_blk, 0)


def kernel(x, edge_index, edge_attr, W_src, W_dst, b):
    wa = W_src[:D_IN]
    wb = W_src[D_IN:]
    xa_f, nh_f = _node_proj(x, wa, W_dst)
    ea_p = _edge_proj(edge_attr, wb)
    eidx = edge_index.reshape(2, NB, B).transpose(1, 0, 2).reshape(-1)
    out_k = _sc_attention(
        xa_f.reshape(NCH * N, CH),
        nh_f.reshape(NCH * N, CH),
        ea_p,
        eidx,
        b,
    )
    return out_k.reshape(N, H, D_OUT)


# speedup vs baseline: 1.2493x; 1.1977x over previous
import functools

import jax
import jax.numpy as jnp
from jax import lax
from jax.experimental import pallas as pl
from jax.experimental.pallas import tpu as pltpu
from jax.experimental.pallas import tpu_sc as plsc

N = 10000
E = 320000
D_IN = 128
H = 4
D_OUT = 64
HD = H * D_OUT
CH = 64
NCH = HD // CH
NEG_SLOPE = 0.2

B = 80
NB = E // B
NBT = NB // 16
NPAIR = NBT // 2
RB = 40
NRB = N // RB
L = 16


def _node_proj_body(x_ref, wa_ref, wd_ref, xa_ref, nh_ref):
    xb = x_ref[...]
    pa = jnp.dot(xb, wa_ref[...], preferred_element_type=jnp.float32)
    pd = jnp.dot(xb, wd_ref[...], preferred_element_type=jnp.float32)
    for c in range(NCH):
        xa_ref[c] = pa[:, c * CH:(c + 1) * CH]
        nh_ref[c] = pd[:, c * CH:(c + 1) * CH]


def _node_proj(x, wa, wd):
    rbn = 2000
    return pl.pallas_call(
        _node_proj_body,
        grid=(N // rbn,),
        in_specs=[
            pl.BlockSpec((rbn, D_IN), lambda r: (r, 0)),
            pl.BlockSpec((D_IN, HD), lambda r: (0, 0)),
            pl.BlockSpec((D_IN, HD), lambda r: (0, 0)),
        ],
        out_specs=[
            pl.BlockSpec((NCH, rbn, CH), lambda r: (0, r, 0)),
            pl.BlockSpec((NCH, rbn, CH), lambda r: (0, r, 0)),
        ],
        out_shape=[
            jax.ShapeDtypeStruct((NCH, N, CH), jnp.float32),
            jax.ShapeDtypeStruct((NCH, N, CH), jnp.float32),
        ],
    )(x, wa, wd)


def _edge_proj_body(ea_ref, w_ref, out_ref):
    prod = jnp.dot(ea_ref[...], w_ref[...],
                   preferred_element_type=jnp.float32)
    out_ref[0] = prod[:, :HD // 2]
    out_ref[1] = prod[:, HD // 2:]


def _edge_proj(edge_attr, wb):
    eb = 2000
    return pl.pallas_call(
        _edge_proj_body,
        grid=(E // eb,),
        in_specs=[
            pl.BlockSpec((eb, D_IN), lambda e: (e, 0)),
            pl.BlockSpec((D_IN, HD), lambda e: (0, 0)),
        ],
        out_specs=pl.BlockSpec((2, eb, HD // 2), lambda e: (0, e, 0)),
        out_shape=jax.ShapeDtypeStruct((2, E, HD // 2), jnp.float32),
    )(edge_attr, wb)


_mesh = plsc.VectorSubcoreMesh(core_axis_name="c", subcore_axis_name="s")


@functools.partial(
    pl.kernel,
    out_type=jax.ShapeDtypeStruct((N, HD), jnp.float32),
    mesh=_mesh,
    compiler_params=pltpu.CompilerParams(use_tc_tiling_on_sc=False),
    scratch_types=[
        pltpu.VMEM((2, 2 * B), jnp.int32),
        pltpu.VMEM((2, B), jnp.int32),
        pltpu.VMEM((2, B), jnp.int32),
        pltpu.VMEM((2, B), jnp.int32),
        pltpu.VMEM((2, B, CH), jnp.float32),
        pltpu.VMEM((2, B, CH), jnp.float32),
        pltpu.VMEM((2, B, CH), jnp.float32),
        pltpu.VMEM((RB, CH), jnp.float32),
        pltpu.VMEM((RB, CH), jnp.float32),
        pltpu.VMEM((RB, CH), jnp.float32),
        pltpu.VMEM((CH,), jnp.float32),
        pltpu.VMEM_SHARED((N, CH), jnp.float32),
        pltpu.VMEM_SHARED((N, CH), jnp.float32),
        pltpu.SemaphoreType.DMA,
        pltpu.SemaphoreType.DMA,
        pltpu.SemaphoreType.DMA,
        pltpu.SemaphoreType.DMA,
        pltpu.SemaphoreType.DMA,
        pltpu.SemaphoreType.DMA,
    ],
)
def _sc_attention(xa_hbm, nh_hbm, ea_hbm, eidx_hbm, b_hbm, out_hbm,
                  idxb, gs, gd, didx, xab, nhb, eab,
                  dbe, dbw, ob, bb, esum_sh, wsum_sh,
                  semi0, semi1, semg0, semg1, sc0, sc1):
    cid = lax.axis_index("c")
    sid = lax.axis_index("s")
    semi = (semi0, semi1)
    semg = (semg0, semg1)
    sc = (sc0, sc1)

    def start_idx(slot, t):
        bidx = sid + t * 16
        pltpu.async_copy(eidx_hbm.at[pl.ds(bidx * 2 * B, 2 * B)],
                         idxb.at[slot], semi[slot])

    def wait_idx(slot):
        pltpu.make_async_copy(eidx_hbm.at[pl.ds(0, 2 * B)],
                              idxb.at[slot], semi[slot]).wait()

    def adjust(slot, coff):
        for g in range(B // L):
            sl = pl.ds(g * L, L)
            s16 = idxb[slot, pl.ds(g * L, L)]
            d16 = idxb[slot, pl.ds(B + g * L, L)]
            gs[slot, sl] = s16 + coff
            gd[slot, sl] = d16 + coff
            didx[slot, sl] = d16

    def start_gathers(slot, t, k):
        bidx = sid + t * 16
        e0 = cid * E + bidx * B
        pltpu.async_copy(xa_hbm.at[gs.at[slot]], xab.at[slot], semg[slot])
        pltpu.async_copy(nh_hbm.at[gd.at[slot]], nhb.at[slot], semg[slot])
        pltpu.async_copy(ea_hbm.at[pl.ds(e0, B), pl.ds(k * CH, CH)],
                         eab.at[slot], semg[slot])

    def wait_gathers(slot):
        pltpu.make_async_copy(xa_hbm.at[pl.ds(0, B)], xab.at[slot],
                              semg[slot]).wait()
        pltpu.make_async_copy(xa_hbm.at[pl.ds(0, B)], nhb.at[slot],
                              semg[slot]).wait()
        pltpu.make_async_copy(xa_hbm.at[pl.ds(0, B)], eab.at[slot],
                              semg[slot]).wait()

    def compute(slot):
        @plsc.parallel_loop(0, B, step=1, unroll=4)
        def _crow(r):
            for f in range(CH // L):
                sl = pl.ds(f * L, L)
                m = xab[slot, r, sl] + eab[slot, r, sl]
                t = m + nhb[slot, r, sl]
                lr = jnp.maximum(t, t * NEG_SLOPE)
                p = jnp.exp(lr)
                nhb[slot, r, sl] = p
                eab[slot, r, sl] = m * p

    def start_scatter(slot):
        pltpu.async_copy(nhb.at[slot], esum_sh.at[didx.at[slot]],
                         sc[slot], add=True)
        pltpu.async_copy(eab.at[slot], wsum_sh.at[didx.at[slot]],
                         sc[slot], add=True)

    def wait_scatter(slot):
        pltpu.make_async_copy(xa_hbm.at[pl.ds(0, B)], nhb.at[slot],
                              sc[slot]).wait()
        pltpu.make_async_copy(xa_hbm.at[pl.ds(0, B)], eab.at[slot],
                              sc[slot]).wait()

    for k in range(2):
        c = cid * 2 + k
        coff = c * N

        plsc.subcore_barrier()

        def _zrow(r, carry):
            for f in range(CH // L):
                ob[r, pl.ds(f * L, L)] = jnp.zeros((L,), jnp.float32)
            return carry
        lax.fori_loop(0, RB, _zrow, 0)

        def _zero_blk(j, carry):
            blk = sid + j * 16

            @pl.when(blk < NRB)
            def _():
                r0 = blk * RB
                pltpu.sync_copy(ob, esum_sh.at[pl.ds(r0, RB)])
                pltpu.sync_copy(ob, wsum_sh.at[pl.ds(r0, RB)])
            return carry
        lax.fori_loop(0, (NRB + 15) // 16, _zero_blk, 0)

        pltpu.sync_copy(b_hbm.at[pl.ds(c * CH, CH)], bb)
        plsc.subcore_barrier()

        start_idx(0, 0)
        start_idx(1, 1)
        wait_idx(0)
        adjust(0, coff)
        start_gathers(0, 0, k)

        def _pair(j, carry):
            for b in (0, 1):
                t = 2 * j + b
                s = 1 - b

                @pl.when(t <= NBT - 2)
                def _():
                    wait_idx(s)

                    @pl.when(t >= 1)
                    def _():
                        wait_scatter(s)
                    adjust(s, coff)
                    start_gathers(s, t + 1, k)

                    @pl.when(t <= NBT - 3)
                    def _():
                        start_idx(b, t + 2)
                wait_gathers(b)
                compute(b)
                start_scatter(b)
            return carry
        lax.fori_loop(0, NPAIR, _pair, 0)
        wait_scatter(0)
        wait_scatter(1)

        plsc.subcore_barrier()

        def _flush_blk(j, carry):
            blk = sid + j * 16

            @pl.when(blk < NRB)
            def _():
                r0 = blk * RB
                pltpu.sync_copy(esum_sh.at[pl.ds(r0, RB)], dbe)
                pltpu.sync_copy(wsum_sh.at[pl.ds(r0, RB)], dbw)

                @plsc.parallel_loop(0, RB, step=1, unroll=4)
                def _drow(r):
                    for f in range(CH // L):
                        sl = pl.ds(f * L, L)
                        es = dbe[r, sl]
                        ws = dbw[r, sl]
                        val = jnp.where(es > 0.0, ws / es, 0.0) + bb[sl]
                        ob[r, sl] = val
                pltpu.sync_copy(
                    ob, out_hbm.at[pl.ds(r0, RB), pl.ds(c * CH, CH)])
            return carry
        lax.fori_loop(0, (NRB + 15) // 16, _flush_blk, 0)


def kernel(x, edge_index, edge_attr, W_src, W_dst, b):
    wa = W_src[:D_IN]
    wb = W_src[D_IN:]
    xa_f, nh_f = _node_proj(x, wa, W_dst)
    ea_p = _edge_proj(edge_attr, wb)
    eidx = edge_index.reshape(2, NB, B).transpose(1, 0, 2).reshape(-1)
    out_k = _sc_attention(
        xa_f.reshape(NCH * N, CH),
        nh_f.reshape(NCH * N, CH),
        ea_p.reshape(2 * E, HD // 2),
        eidx,
        b,
    )
    return out_k.reshape(N, H, D_OUT)
